# baseline (speedup 1.0000x reference)
import functools

import jax
import jax.numpy as jnp
from jax import lax
from jax.experimental import pallas as pl
from jax.experimental.pallas import tpu as pltpu
from jax.experimental.pallas import tpu_sc as plsc

B = 32
D = 512
V = 50257

_ROWS_PER_WORKER = 8
_ACTIVE_WORKERS = B // _ROWS_PER_WORKER

_sc_mesh = plsc.VectorSubcoreMesh(core_axis_name="c", subcore_axis_name="s")


@functools.partial(
    pl.kernel,
    out_type=jax.ShapeDtypeStruct((B, D), jnp.float32),
    mesh=_sc_mesh,
    scratch_types=[
        pltpu.VMEM((_ROWS_PER_WORKER,), jnp.int32),
        pltpu.VMEM((_ROWS_PER_WORKER, D), jnp.float32),
        pltpu.SemaphoreType.DMA,
    ],
)
def _sc_gather(emb_hbm, ids_hbm, out_hbm, idx_v, rows_v, sem):
    info = plsc.get_sparse_core_info()
    nc = info.num_cores
    wid = lax.axis_index("s") * nc + lax.axis_index("c")

    @pl.when(wid < _ACTIVE_WORKERS)
    def _():
        base = wid * _ROWS_PER_WORKER
        pltpu.sync_copy(ids_hbm.at[pl.ds(base, _ROWS_PER_WORKER)], idx_v)
        pltpu.async_copy(emb_hbm.at[idx_v], rows_v, sem).wait()
        pltpu.sync_copy(rows_v, out_hbm.at[pl.ds(base, _ROWS_PER_WORKER)])


_VB = 2048
_NV = (V + _VB - 1) // _VB


def _mm_body(x_ref, w_ref, b_ref, o_ref):
    o_ref[...] = (
        jnp.dot(x_ref[...], w_ref[...], preferred_element_type=jnp.float32)
        + b_ref[...]
    )


def _tc_project(x, W, b2d):
    return pl.pallas_call(
        _mm_body,
        grid=(_NV,),
        in_specs=[
            pl.BlockSpec((B, D), lambda v: (0, 0)),
            pl.BlockSpec((D, _VB), lambda v: (0, v)),
            pl.BlockSpec((1, _VB), lambda v: (0, v)),
        ],
        out_specs=pl.BlockSpec((B, _VB), lambda v: (0, v)),
        out_shape=jax.ShapeDtypeStruct((B, V), jnp.float32),
        compiler_params=pltpu.CompilerParams(
            dimension_semantics=("arbitrary",),
        ),
    )(x, W, b2d)


def kernel(input_ids, emb, W, b):
    ids = input_ids.reshape(B).astype(jnp.int32)
    x = _sc_gather(emb, ids)
    logits = _tc_project(x, W, b.reshape(1, V))
    return logits.reshape(B, 1, V)
